# Initial kernel scaffold; baseline (speedup 1.0000x reference)
import jax
import jax.numpy as jnp
from jax.experimental import pallas as pl


def kernel(x, wkv, wgate, ape, norm_weight, rope_cos, rope_sin, cache, slot_mapping):
    raise NotImplementedError("write your pallas kernel here")


# speedup vs baseline: 2.0265x; 2.0265x over previous
import functools

import jax
import jax.numpy as jnp
from jax import lax
from jax.experimental import pallas as pl
from jax.experimental.pallas import tpu as pltpu
from jax.experimental.pallas import tpu_sc as plsc

DIM = 4096
HEAD_DIM = 512
ROPE = 64
R = 4
COFF = 2
T = 8192
TC = T // R
SLOTS = 16384
EPS = 1e-6
NKV = COFF * HEAD_DIM

TB = 256
CB = TB // R


def _prolog_body(x_ref, w_ref, ape_ref, nw_ref, cos_ref, sin_ref, kv_ref):
    xb = x_ref[...].astype(jnp.bfloat16)
    y = lax.dot_general(xb, w_ref[...], (((1,), (1,)), ((), ())),
                        preferred_element_type=jnp.float32)
    kvp = y[:, :NKV]
    gate = jax.nn.sigmoid(y[:, NKV:])
    h = (kvp * gate).astype(jnp.bfloat16)

    r_i = lax.broadcasted_iota(jnp.int32, (CB, TB), 0)
    c_i = lax.broadcasted_iota(jnp.int32, (CB, TB), 1)
    A = (c_i // R == r_i).astype(jnp.bfloat16)
    hc = lax.dot_general(A, h, (((1,), (0,)), ((), ())),
                         preferred_element_type=jnp.float32)
    ape_sum = jnp.sum(ape_ref[...], axis=0, keepdims=True)
    hc = hc + ape_sum

    h1 = hc[:, :HEAD_DIM]
    h2 = hc[:, HEAD_DIM:]
    v1 = jnp.mean(h1 * h1, axis=1, keepdims=True)
    v2 = jnp.mean(h2 * h2, axis=1, keepdims=True)
    hn = jnp.concatenate([h1 * lax.rsqrt(v1 + EPS),
                          h2 * lax.rsqrt(v2 + EPS)], axis=1) * nw_ref[...]

    lane = lax.broadcasted_iota(jnp.int32, (CB, NKV), 1)
    km = lane % HEAD_DIM
    in_rope = km >= HEAD_DIM - ROPE
    sign = jnp.where(km < HEAD_DIM - ROPE // 2, -1.0, 1.0)

    j_i = lax.broadcasted_iota(jnp.int32, (NKV, NKV), 0)
    k_i = lax.broadcasted_iota(jnp.int32, (NKV, NKV), 1)
    kk = k_i % HEAD_DIM
    P = (((kk >= HEAD_DIM - ROPE) & (kk < HEAD_DIM - ROPE // 2)
          & (j_i == k_i + ROPE // 2))
         | ((kk >= HEAD_DIM - ROPE // 2) & (j_i == k_i - ROPE // 2)))
    hs = lax.dot_general(hn.astype(jnp.bfloat16), P.astype(jnp.bfloat16),
                         (((1,), (0,)), ((), ())),
                         preferred_element_type=jnp.float32)

    r_e = lax.broadcasted_iota(jnp.int32, (2 * ROPE, NKV), 0)
    k_e = lax.broadcasted_iota(jnp.int32, (2 * ROPE, NKV), 1)
    E = ((k_e % HEAD_DIM >= HEAD_DIM - ROPE)
         & (r_e == k_e % HEAD_DIM - (HEAD_DIM - ROPE))).astype(jnp.float32)
    cosf = lax.dot_general(cos_ref[...], E, (((1,), (0,)), ((), ())),
                           preferred_element_type=jnp.float32)
    sinf = lax.dot_general(sin_ref[...], E, (((1,), (0,)), ((), ())),
                           preferred_element_type=jnp.float32)
    cosf = jnp.where(in_rope, cosf, 1.0)
    sinf = jnp.where(in_rope, sinf * sign, 0.0)

    kv_ref[...] = hn * cosf + hs * sinf


def _compute_kv(x, w2, ape8, nw2, cos_p, sin_p):
    grid = T // TB
    return pl.pallas_call(
        _prolog_body,
        grid=(grid,),
        in_specs=[
            pl.BlockSpec((TB, DIM), lambda i: (i, 0)),
            pl.BlockSpec((2 * NKV, DIM), lambda i: (0, 0)),
            pl.BlockSpec((8, NKV), lambda i: (0, 0)),
            pl.BlockSpec((1, NKV), lambda i: (0, 0)),
            pl.BlockSpec((CB, 2 * ROPE), lambda i: (i, 0)),
            pl.BlockSpec((CB, 2 * ROPE), lambda i: (i, 0)),
        ],
        out_specs=pl.BlockSpec((CB, NKV), lambda i: (i, 0)),
        out_shape=jax.ShapeDtypeStruct((TC, NKV), jnp.float32),
    )(x, w2, ape8, nw2, cos_p, sin_p)


NWORK = 32
STRIPE = SLOTS // NWORK
SCHUNK = 1024


CCHUNK = 64


def _scatter_body(cache_hbm, kv_hbm, slots_hbm, out_hbm,
                  slot_vmem, row_vmem, buf_vmem):
    c = lax.axis_index("c")
    s = lax.axis_index("s")
    wid = s * 2 + c
    base = wid * STRIPE

    def copy_chunk(i, carry):
        pltpu.sync_copy(cache_hbm.at[pl.ds(base + i * CCHUNK, CCHUNK)],
                        buf_vmem)
        pltpu.sync_copy(buf_vmem,
                        out_hbm.at[pl.ds(base + i * CCHUNK, CCHUNK)])
        return carry

    lax.fori_loop(0, STRIPE // CCHUNK, copy_chunk, 0)

    pltpu.sync_copy(slots_hbm, slot_vmem)

    def body(k, carry):
        v = slot_vmem[pl.ds(k * 16, 16)]
        m = (v >= base) & (v < base + STRIPE)
        nh = plsc.all_reduce_population_count(m)[0]

        @pl.when(nh > 0)
        def _():
            for j in range(16):
                sl = v[j]
                hit = (sl >= base) & (sl < base + STRIPE)

                @pl.when(hit)
                def _():
                    pltpu.sync_copy(kv_hbm.at[k * 16 + j], row_vmem)
                    pltpu.sync_copy(row_vmem, out_hbm.at[sl])

        return carry

    lax.fori_loop(0, TC // 16, body, 0)


def _scatter(cache, kv, slot_mapping):
    mesh = plsc.VectorSubcoreMesh(core_axis_name="c", subcore_axis_name="s")
    f = pl.kernel(
        _scatter_body,
        out_type=jax.ShapeDtypeStruct((SLOTS, NKV), jnp.float32),
        mesh=mesh,
        compiler_params=pltpu.CompilerParams(needs_layout_passes=False),
        scratch_types=[
            pltpu.VMEM((TC,), jnp.int32),
            pltpu.VMEM((NKV,), jnp.float32),
            pltpu.VMEM((CCHUNK, NKV), jnp.float32),
        ],
    )
    return f(cache, kv, slot_mapping)


def kernel(x, wkv, wgate, ape, norm_weight, rope_cos, rope_sin, cache,
           slot_mapping):
    w2 = jnp.concatenate([wkv, wgate], axis=0).astype(jnp.bfloat16)
    ape8 = jnp.pad(ape, ((0, 8 - R), (0, 0)))
    nw2 = jnp.concatenate([norm_weight, norm_weight]).reshape(1, NKV)
    cos_p = jnp.pad(rope_cos, ((0, 0), (0, 2 * ROPE - ROPE)))
    sin_p = jnp.pad(rope_sin, ((0, 0), (0, 2 * ROPE - ROPE)))
    kv = _compute_kv(x, w2, ape8, nw2, cos_p, sin_p)
    return _scatter(cache, kv, slot_mapping.astype(jnp.int32))
